# Initial kernel scaffold; baseline (speedup 1.0000x reference)
import jax
import jax.numpy as jnp
from jax.experimental import pallas as pl


def kernel(x, edge_index, W_pre, b_pre, W_l1, b_l1, W_r1, W_l2, b_l2, W_r2, W_post, b_post):
    raise NotImplementedError("write your pallas kernel here")


# speedup vs baseline: 6.2682x; 6.2682x over previous
import jax
import jax.numpy as jnp
from jax import lax
from jax.experimental import pallas as pl
from jax.experimental.pallas import tpu as pltpu
from jax.experimental.pallas import tpu_sc as plsc

N_NODES = 10000
N_EDGES = 320000
D = 128
NC = 2
NS = 16
NW = NC * NS
CHUNK = 128
NCHUNKS = N_EDGES // CHUNK
ROWS_PER_TILE = N_NODES // NS
ZROWS = 125
CW = 16

_SC_PARAMS = pltpu.CompilerParams(use_tc_tiling_on_sc=False)


def _mesh():
    return plsc.VectorSubcoreMesh(core_axis_name="c", subcore_axis_name="s")


def _sc_agg_body(h_hbm, src_hbm, dst_hbm, out_hbm, srcv, dstv, rows, zbuf,
                 acc_sh, sem):
    c = lax.axis_index("c")
    s = lax.axis_index("s")
    wid = c * NS + s

    zero16 = jnp.zeros((16,), jnp.float32)
    def _zb(i, carry):
        zbuf[i // 8, pl.ds((i % 8) * 16, 16)] = zero16
        return carry
    lax.fori_loop(0, ZROWS * 8, _zb, 0)
    for i in range(5):
        pltpu.sync_copy(zbuf, acc_sh.at[pl.ds(s * ROWS_PER_TILE + i * ZROWS, ZROWS)])
    plsc.subcore_barrier()

    nloc = (NCHUNKS - wid + NW - 1) // NW
    def _edge(i, carry):
        base = (wid + i * NW) * CHUNK
        pltpu.sync_copy(src_hbm.at[pl.ds(base, CHUNK)], srcv)
        pltpu.sync_copy(dst_hbm.at[pl.ds(base, CHUNK)], dstv)
        pltpu.async_copy(h_hbm.at[srcv], rows, sem).wait()
        pltpu.sync_copy(rows, acc_sh.at[dstv], add=True)
        return carry
    lax.fori_loop(0, nloc, _edge, 0)
    plsc.subcore_barrier()

    pltpu.sync_copy(acc_sh.at[pl.ds(s * ROWS_PER_TILE, ROWS_PER_TILE)],
                    out_hbm.at[c, pl.ds(s * ROWS_PER_TILE, ROWS_PER_TILE)])


def _make_sc_agg():
    return pl.kernel(
        _sc_agg_body,
        out_type=[jax.ShapeDtypeStruct((NC, N_NODES, D), jnp.float32)],
        mesh=_mesh(),
        scratch_types=[
            pltpu.VMEM((CHUNK,), jnp.int32),
            pltpu.VMEM((CHUNK,), jnp.int32),
            pltpu.VMEM((CHUNK, D), jnp.float32),
            pltpu.VMEM((ZROWS, D), jnp.float32),
            pltpu.VMEM_SHARED((N_NODES, D), jnp.float32),
            pltpu.SemaphoreType.DMA,
        ],
        compiler_params=_SC_PARAMS,
    )


def _sc_count_body(dst_hbm, out_hbm, dstv, onesv, zcnt, cnt_sh):
    c = lax.axis_index("c")
    s = lax.axis_index("s")
    wid = c * NS + s

    zero16 = jnp.zeros((16,), jnp.float32)
    one16 = jnp.ones((16,), jnp.float32)
    def _fill(i, carry):
        onesv[i, :] = one16
        return carry
    lax.fori_loop(0, CHUNK, _fill, 0)
    def _zc(i, carry):
        zcnt[i, :] = zero16
        return carry
    lax.fori_loop(0, ROWS_PER_TILE, _zc, 0)
    pltpu.sync_copy(zcnt, cnt_sh.at[pl.ds(s * ROWS_PER_TILE, ROWS_PER_TILE)])
    plsc.subcore_barrier()

    nloc = (NCHUNKS - wid + NW - 1) // NW
    def _edge(i, carry):
        base = (wid + i * NW) * CHUNK
        pltpu.sync_copy(dst_hbm.at[pl.ds(base, CHUNK)], dstv)
        pltpu.sync_copy(onesv, cnt_sh.at[dstv], add=True)
        return carry
    lax.fori_loop(0, nloc, _edge, 0)
    plsc.subcore_barrier()

    pltpu.sync_copy(cnt_sh.at[pl.ds(s * ROWS_PER_TILE, ROWS_PER_TILE)],
                    out_hbm.at[c, pl.ds(s * ROWS_PER_TILE, ROWS_PER_TILE)])


def _make_sc_count():
    return pl.kernel(
        _sc_count_body,
        out_type=[jax.ShapeDtypeStruct((NC, N_NODES, CW), jnp.float32)],
        mesh=_mesh(),
        scratch_types=[
            pltpu.VMEM((CHUNK,), jnp.int32),
            pltpu.VMEM((CHUNK, CW), jnp.float32),
            pltpu.VMEM((ROWS_PER_TILE, CW), jnp.float32),
            pltpu.VMEM_SHARED((N_NODES, CW), jnp.float32),
        ],
        compiler_params=_SC_PARAMS,
    )


def _matT(a, w):
    return lax.dot_general(a, w, (((1,), (1,)), ((), ())),
                           preferred_element_type=jnp.float32)


def _tc_pre_body(x_ref, w_ref, b_ref, o_ref):
    o_ref[...] = _matT(x_ref[...], w_ref[...]) + b_ref[...]


def _tc_combine_body(p_ref, cnt_ref, h_ref, wl_ref, bl_ref, wr_ref, o_ref):
    agg = p_ref[0] + p_ref[1]
    cnt = cnt_ref[0, :, 0:1] + cnt_ref[1, :, 0:1]
    mean = agg / jnp.maximum(cnt, 1.0)
    o = _matT(mean, wl_ref[...]) + bl_ref[...] + _matT(h_ref[...], wr_ref[...])
    o_ref[...] = jnp.maximum(o, 0.0)


def _tc_final_body(p_ref, cnt_ref, h_ref, wl_ref, bl_ref, wr_ref,
                   wpost_ref, bpost_ref, o_ref):
    agg = p_ref[0] + p_ref[1]
    cnt = cnt_ref[0, :, 0:1] + cnt_ref[1, :, 0:1]
    mean = agg / jnp.maximum(cnt, 1.0)
    h2 = _matT(mean, wl_ref[...]) + bl_ref[...] + _matT(h_ref[...], wr_ref[...])
    h2 = jnp.maximum(h2, 0.0)
    logits = _matT(h2, wpost_ref[...]) + bpost_ref[...]
    m = jnp.max(logits, axis=1, keepdims=True)
    lse = jnp.log(jnp.sum(jnp.exp(logits - m), axis=1, keepdims=True)) + m
    o_ref[...] = logits - lse


@jax.jit
def kernel(x, edge_index, W_pre, b_pre, W_l1, b_l1, W_r1, W_l2, b_l2, W_r2,
           W_post, b_post):
    n_class = W_post.shape[0]
    src = edge_index[0]
    dst = edge_index[1]

    h0 = pl.pallas_call(
        _tc_pre_body,
        out_shape=jax.ShapeDtypeStruct((N_NODES, D), jnp.float32),
    )(x, W_pre, b_pre.reshape(1, D))

    (cnt,) = _make_sc_count()(dst)

    sc_agg = _make_sc_agg()
    (p1,) = sc_agg(h0, src, dst)

    h1 = pl.pallas_call(
        _tc_combine_body,
        out_shape=jax.ShapeDtypeStruct((N_NODES, D), jnp.float32),
    )(p1, cnt, h0, W_l1, b_l1.reshape(1, D), W_r1)

    (p2,) = sc_agg(h1, src, dst)

    out = pl.pallas_call(
        _tc_final_body,
        out_shape=jax.ShapeDtypeStruct((N_NODES, n_class), jnp.float32),
    )(p2, cnt, h1, W_l2, b_l2.reshape(1, D), W_r2, W_post,
      b_post.reshape(1, n_class))
    return out


# speedup vs baseline: 8.4255x; 1.3442x over previous
import jax
import jax.numpy as jnp
from jax import lax
from jax.experimental import pallas as pl
from jax.experimental.pallas import tpu as pltpu
from jax.experimental.pallas import tpu_sc as plsc

N_NODES = 10000
N_EDGES = 320000
D = 128
NC = 2
NS = 16
NW = NC * NS
CHUNK = 40
NCHUNKS = N_EDGES // CHUNK
NLOC = NCHUNKS // NW
NBUF = 2
NROUNDS = NLOC // NBUF
ROWS_PER_TILE = N_NODES // NS
ZROWS = 125
CW = 16

_SC_PARAMS = pltpu.CompilerParams(use_tc_tiling_on_sc=False)


def _mesh():
    return plsc.VectorSubcoreMesh(core_axis_name="c", subcore_axis_name="s")


def _sc_agg_body(h_hbm, src_hbm, dst_hbm, out_hbm, idxs, idxd, rows, zbuf,
                 acc_sh, isem, *sems):
    gsem = sems[:NBUF]
    ssem = sems[NBUF:]
    c = lax.axis_index("c")
    s = lax.axis_index("s")
    wid = c * NS + s

    idx_g = pltpu.async_copy(src_hbm.at[wid], idxs, isem)
    idx_d = pltpu.async_copy(dst_hbm.at[wid], idxd, isem)

    zero16 = jnp.zeros((16,), jnp.float32)
    def _zb(i, carry):
        zbuf[i // 8, pl.ds((i % 8) * 16, 16)] = zero16
        return carry
    lax.fori_loop(0, ZROWS * 8, _zb, 0)
    for i in range(5):
        pltpu.sync_copy(zbuf, acc_sh.at[pl.ds(s * ROWS_PER_TILE + i * ZROWS, ZROWS)])
    idx_g.wait()
    idx_d.wait()
    plsc.subcore_barrier()

    def _gather(b, j):
        pltpu.async_copy(h_hbm.at[idxs.at[j]], rows.at[b], gsem[b])

    def _wait_gather(b):
        pltpu.make_async_copy(h_hbm.at[idxs.at[0]], rows.at[b], gsem[b]).wait()

    def _scatter(b, j):
        pltpu.async_copy(rows.at[b], acc_sh.at[idxd.at[j]], ssem[b], add=True)

    def _wait_scatter(b):
        pltpu.make_async_copy(rows.at[b], acc_sh.at[idxd.at[0]], ssem[b]).wait()

    for b in range(NBUF):
        _gather(b, b)
    def _round(r, carry):
        j0 = r * NBUF
        for b in range(NBUF):
            _wait_gather(b)
            _scatter(b, j0 + b)
        for b in range(NBUF):
            _wait_scatter(b)
            _gather(b, j0 + NBUF + b)
        return carry
    lax.fori_loop(0, NROUNDS - 1, _round, 0)
    j0 = (NROUNDS - 1) * NBUF
    for b in range(NBUF):
        _wait_gather(b)
        _scatter(b, j0 + b)
    for b in range(NBUF):
        _wait_scatter(b)
    plsc.subcore_barrier()

    pltpu.sync_copy(acc_sh.at[pl.ds(s * ROWS_PER_TILE, ROWS_PER_TILE)],
                    out_hbm.at[c, pl.ds(s * ROWS_PER_TILE, ROWS_PER_TILE)])


def _make_sc_agg():
    return pl.kernel(
        _sc_agg_body,
        out_type=[jax.ShapeDtypeStruct((NC, N_NODES, D), jnp.float32)],
        mesh=_mesh(),
        scratch_types=[
            pltpu.VMEM((NLOC, CHUNK), jnp.int32),
            pltpu.VMEM((NLOC, CHUNK), jnp.int32),
            pltpu.VMEM((NBUF, CHUNK, D), jnp.float32),
            pltpu.VMEM((ZROWS, D), jnp.float32),
            pltpu.VMEM_SHARED((N_NODES, D), jnp.float32),
            pltpu.SemaphoreType.DMA,
        ] + [pltpu.SemaphoreType.DMA] * (2 * NBUF),
        compiler_params=_SC_PARAMS,
    )


def _sc_count_body(dst_hbm, out_hbm, idxd, onesv, zcnt, cnt_sh, isem):
    c = lax.axis_index("c")
    s = lax.axis_index("s")
    wid = c * NS + s

    idx_d = pltpu.async_copy(dst_hbm.at[wid], idxd, isem)
    zero16 = jnp.zeros((16,), jnp.float32)
    one16 = jnp.ones((16,), jnp.float32)
    def _fill(i, carry):
        onesv[i, :] = one16
        return carry
    lax.fori_loop(0, CHUNK, _fill, 0)
    def _zc(i, carry):
        zcnt[i, :] = zero16
        return carry
    lax.fori_loop(0, ROWS_PER_TILE, _zc, 0)
    pltpu.sync_copy(zcnt, cnt_sh.at[pl.ds(s * ROWS_PER_TILE, ROWS_PER_TILE)])
    idx_d.wait()
    plsc.subcore_barrier()

    def _edge(j, carry):
        pltpu.sync_copy(onesv, cnt_sh.at[idxd.at[j]], add=True)
        return carry
    lax.fori_loop(0, NLOC, _edge, 0)
    plsc.subcore_barrier()

    pltpu.sync_copy(cnt_sh.at[pl.ds(s * ROWS_PER_TILE, ROWS_PER_TILE)],
                    out_hbm.at[c, pl.ds(s * ROWS_PER_TILE, ROWS_PER_TILE)])


def _make_sc_count():
    return pl.kernel(
        _sc_count_body,
        out_type=[jax.ShapeDtypeStruct((NC, N_NODES, CW), jnp.float32)],
        mesh=_mesh(),
        scratch_types=[
            pltpu.VMEM((NLOC, CHUNK), jnp.int32),
            pltpu.VMEM((CHUNK, CW), jnp.float32),
            pltpu.VMEM((ROWS_PER_TILE, CW), jnp.float32),
            pltpu.VMEM_SHARED((N_NODES, CW), jnp.float32),
            pltpu.SemaphoreType.DMA,
        ],
        compiler_params=_SC_PARAMS,
    )


def _matT(a, w):
    return lax.dot_general(a, w, (((1,), (1,)), ((), ())),
                           preferred_element_type=jnp.float32)


def _tc_pre_body(x_ref, w_ref, b_ref, o_ref):
    o_ref[...] = _matT(x_ref[...], w_ref[...]) + b_ref[...]


def _tc_combine_body(p_ref, cnt_ref, h_ref, wl_ref, bl_ref, wr_ref, o_ref):
    agg = p_ref[0] + p_ref[1]
    cnt = cnt_ref[0, :, 0:1] + cnt_ref[1, :, 0:1]
    mean = agg / jnp.maximum(cnt, 1.0)
    o = _matT(mean, wl_ref[...]) + bl_ref[...] + _matT(h_ref[...], wr_ref[...])
    o_ref[...] = jnp.maximum(o, 0.0)


def _tc_final_body(p_ref, cnt_ref, h_ref, wl_ref, bl_ref, wr_ref,
                   wpost_ref, bpost_ref, o_ref):
    agg = p_ref[0] + p_ref[1]
    cnt = cnt_ref[0, :, 0:1] + cnt_ref[1, :, 0:1]
    mean = agg / jnp.maximum(cnt, 1.0)
    h2 = _matT(mean, wl_ref[...]) + bl_ref[...] + _matT(h_ref[...], wr_ref[...])
    h2 = jnp.maximum(h2, 0.0)
    logits = _matT(h2, wpost_ref[...]) + bpost_ref[...]
    m = jnp.max(logits, axis=1, keepdims=True)
    lse = jnp.log(jnp.sum(jnp.exp(logits - m), axis=1, keepdims=True)) + m
    o_ref[...] = logits - lse


@jax.jit
def kernel(x, edge_index, W_pre, b_pre, W_l1, b_l1, W_r1, W_l2, b_l2, W_r2,
           W_post, b_post):
    n_class = W_post.shape[0]
    src = edge_index[0].reshape(NW, NLOC, CHUNK)
    dst = edge_index[1].reshape(NW, NLOC, CHUNK)

    h0 = pl.pallas_call(
        _tc_pre_body,
        out_shape=jax.ShapeDtypeStruct((N_NODES, D), jnp.float32),
    )(x, W_pre, b_pre.reshape(1, D))

    (cnt,) = _make_sc_count()(dst)

    sc_agg = _make_sc_agg()
    (p1,) = sc_agg(h0, src, dst)

    h1 = pl.pallas_call(
        _tc_combine_body,
        out_shape=jax.ShapeDtypeStruct((N_NODES, D), jnp.float32),
    )(p1, cnt, h0, W_l1, b_l1.reshape(1, D), W_r1)

    (p2,) = sc_agg(h1, src, dst)

    out = pl.pallas_call(
        _tc_final_body,
        out_shape=jax.ShapeDtypeStruct((N_NODES, n_class), jnp.float32),
    )(p2, cnt, h1, W_l2, b_l2.reshape(1, D), W_r2, W_post,
      b_post.reshape(1, n_class))
    return out


# speedup vs baseline: 11.7067x; 1.3894x over previous
import jax
import jax.numpy as jnp
from jax import lax
from jax.experimental import pallas as pl
from jax.experimental.pallas import tpu as pltpu
from jax.experimental.pallas import tpu_sc as plsc

N_NODES = 10000
N_EDGES = 320000
D = 128
DH = D // 2
NC = 2
NS = 16
NW = NC * NS

CH = 80
NLOC = N_EDGES // (NS * CH)
NBUF = 5
NROUNDS = NLOC // NBUF

CCH = 80
CNLOC = N_EDGES // (NW * CCH)

ROWS_PER_TILE = N_NODES // NS
ZROWS = 125
CW = 16

_SC_PARAMS = pltpu.CompilerParams(use_tc_tiling_on_sc=False)


def _mesh():
    return plsc.VectorSubcoreMesh(core_axis_name="c", subcore_axis_name="s")


def _sc_agg_body(h_hbm, src_hbm, dst_hbm, out_hbm, idxs, idxd, rows, zbuf,
                 acc_sh, isem, *sems):
    gsem = sems[:NBUF]
    ssem = sems[NBUF:]
    c = lax.axis_index("c")
    s = lax.axis_index("s")

    idx_g = pltpu.async_copy(src_hbm.at[s], idxs, isem)
    idx_d = pltpu.async_copy(dst_hbm.at[s], idxd, isem)

    zero16 = jnp.zeros((16,), jnp.float32)
    def _zb(i, carry):
        zbuf[i // 4, pl.ds((i % 4) * 16, 16)] = zero16
        return carry
    lax.fori_loop(0, ZROWS * 4, _zb, 0)
    for i in range(5):
        pltpu.sync_copy(zbuf, acc_sh.at[pl.ds(s * ROWS_PER_TILE + i * ZROWS, ZROWS)])
    idx_g.wait()
    idx_d.wait()

    cn = c * N_NODES
    def _off(r, carry):
        for j in range(CH // 16):
            sl = pl.ds(j * 16, 16)
            idxs[r, sl] = idxs[r, sl] + cn
        return carry
    lax.fori_loop(0, NLOC, _off, 0)
    plsc.subcore_barrier()

    def _gather(b, j):
        pltpu.async_copy(h_hbm.at[idxs.at[j]], rows.at[b], gsem[b])

    def _wait_gather(b):
        pltpu.make_async_copy(h_hbm.at[idxs.at[0]], rows.at[b], gsem[b]).wait()

    def _scatter(b, j):
        pltpu.async_copy(rows.at[b], acc_sh.at[idxd.at[j]], ssem[b], add=True)

    def _wait_scatter(b):
        pltpu.make_async_copy(rows.at[b], acc_sh.at[idxd.at[0]], ssem[b]).wait()

    for b in range(NBUF):
        _gather(b, b)
    def _round(r, carry):
        j0 = r * NBUF
        for b in range(NBUF):
            _wait_gather(b)
            _scatter(b, j0 + b)
        for b in range(NBUF):
            _wait_scatter(b)
            _gather(b, j0 + NBUF + b)
        return carry
    lax.fori_loop(0, NROUNDS - 1, _round, 0)
    j0 = (NROUNDS - 1) * NBUF
    for b in range(NBUF):
        _wait_gather(b)
        _scatter(b, j0 + b)
    for b in range(NBUF):
        _wait_scatter(b)
    plsc.subcore_barrier()

    pltpu.sync_copy(acc_sh.at[pl.ds(s * ROWS_PER_TILE, ROWS_PER_TILE)],
                    out_hbm.at[c, pl.ds(s * ROWS_PER_TILE, ROWS_PER_TILE)])


def _make_sc_agg():
    return pl.kernel(
        _sc_agg_body,
        out_type=[jax.ShapeDtypeStruct((NC, N_NODES, DH), jnp.float32)],
        mesh=_mesh(),
        scratch_types=[
            pltpu.VMEM((NLOC, CH), jnp.int32),
            pltpu.VMEM((NLOC, CH), jnp.int32),
            pltpu.VMEM((NBUF, CH, DH), jnp.float32),
            pltpu.VMEM((ZROWS, DH), jnp.float32),
            pltpu.VMEM_SHARED((N_NODES, DH), jnp.float32),
            pltpu.SemaphoreType.DMA,
        ] + [pltpu.SemaphoreType.DMA] * (2 * NBUF),
        compiler_params=_SC_PARAMS,
    )


def _sc_count_body(dst_hbm, out_hbm, idxd, onesv, zcnt, cnt_sh, isem):
    c = lax.axis_index("c")
    s = lax.axis_index("s")
    wid = c * NS + s

    idx_d = pltpu.async_copy(dst_hbm.at[wid], idxd, isem)
    zero16 = jnp.zeros((16,), jnp.float32)
    one16 = jnp.ones((16,), jnp.float32)
    def _fill(i, carry):
        onesv[i, :] = one16
        return carry
    lax.fori_loop(0, CCH, _fill, 0)
    def _zc(i, carry):
        zcnt[i, :] = zero16
        return carry
    lax.fori_loop(0, ROWS_PER_TILE, _zc, 0)
    pltpu.sync_copy(zcnt, cnt_sh.at[pl.ds(s * ROWS_PER_TILE, ROWS_PER_TILE)])
    idx_d.wait()
    plsc.subcore_barrier()

    KB = 5
    def _edge(jb, carry):
        for k in range(KB):
            pltpu.async_copy(onesv, cnt_sh.at[idxd.at[jb * KB + k]], isem,
                             add=True)
        for k in range(KB):
            pltpu.make_async_copy(onesv, cnt_sh.at[idxd.at[0]], isem).wait()
        return carry
    lax.fori_loop(0, CNLOC // KB, _edge, 0)
    plsc.subcore_barrier()

    pltpu.sync_copy(cnt_sh.at[pl.ds(s * ROWS_PER_TILE, ROWS_PER_TILE)],
                    out_hbm.at[c, pl.ds(s * ROWS_PER_TILE, ROWS_PER_TILE)])


def _make_sc_count():
    return pl.kernel(
        _sc_count_body,
        out_type=[jax.ShapeDtypeStruct((NC, N_NODES, CW), jnp.float32)],
        mesh=_mesh(),
        scratch_types=[
            pltpu.VMEM((CNLOC, CCH), jnp.int32),
            pltpu.VMEM((CCH, CW), jnp.float32),
            pltpu.VMEM((ROWS_PER_TILE, CW), jnp.float32),
            pltpu.VMEM_SHARED((N_NODES, CW), jnp.float32),
            pltpu.SemaphoreType.DMA,
        ],
        compiler_params=_SC_PARAMS,
    )


def _matT(a, w):
    return lax.dot_general(a, w, (((1,), (1,)), ((), ())),
                           preferred_element_type=jnp.float32)


def _tc_pre_body(x_ref, w_ref, b_ref, o_ref):
    h = _matT(x_ref[...], w_ref[...]) + b_ref[...]
    o_ref[0] = h[:, :DH]
    o_ref[1] = h[:, DH:]


def _tc_combine_body(p_ref, cnt_ref, h_ref, wl_ref, bl_ref, wr_ref, o_ref):
    agg = jnp.concatenate([p_ref[0], p_ref[1]], axis=1)
    cnt = cnt_ref[0, :, 0:1] + cnt_ref[1, :, 0:1]
    mean = agg / jnp.maximum(cnt, 1.0)
    h = jnp.concatenate([h_ref[0], h_ref[1]], axis=1)
    o = _matT(mean, wl_ref[...]) + bl_ref[...] + _matT(h, wr_ref[...])
    o = jnp.maximum(o, 0.0)
    o_ref[0] = o[:, :DH]
    o_ref[1] = o[:, DH:]


def _tc_final_body(p_ref, cnt_ref, h_ref, wl_ref, bl_ref, wr_ref,
                   wpost_ref, bpost_ref, o_ref):
    agg = jnp.concatenate([p_ref[0], p_ref[1]], axis=1)
    cnt = cnt_ref[0, :, 0:1] + cnt_ref[1, :, 0:1]
    mean = agg / jnp.maximum(cnt, 1.0)
    h = jnp.concatenate([h_ref[0], h_ref[1]], axis=1)
    h2 = _matT(mean, wl_ref[...]) + bl_ref[...] + _matT(h, wr_ref[...])
    h2 = jnp.maximum(h2, 0.0)
    logits = _matT(h2, wpost_ref[...]) + bpost_ref[...]
    m = jnp.max(logits, axis=1, keepdims=True)
    lse = jnp.log(jnp.sum(jnp.exp(logits - m), axis=1, keepdims=True)) + m
    o_ref[...] = logits - lse


@jax.jit
def kernel(x, edge_index, W_pre, b_pre, W_l1, b_l1, W_r1, W_l2, b_l2, W_r2,
           W_post, b_post):
    n_class = W_post.shape[0]
    src_a = edge_index[0].reshape(NS, NLOC, CH)
    dst_a = edge_index[1].reshape(NS, NLOC, CH)
    dst_c = edge_index[1].reshape(NW, CNLOC, CCH)

    h0 = pl.pallas_call(
        _tc_pre_body,
        out_shape=jax.ShapeDtypeStruct((NC, N_NODES, DH), jnp.float32),
    )(x, W_pre, b_pre.reshape(1, D))

    (cnt,) = _make_sc_count()(dst_c)

    sc_agg = _make_sc_agg()
    (p1,) = sc_agg(h0.reshape(NC * N_NODES, DH), src_a, dst_a)

    h1 = pl.pallas_call(
        _tc_combine_body,
        out_shape=jax.ShapeDtypeStruct((NC, N_NODES, DH), jnp.float32),
    )(p1, cnt, h0, W_l1, b_l1.reshape(1, D), W_r1)

    (p2,) = sc_agg(h1.reshape(NC * N_NODES, DH), src_a, dst_a)

    out = pl.pallas_call(
        _tc_final_body,
        out_shape=jax.ShapeDtypeStruct((N_NODES, n_class), jnp.float32),
    )(p2, cnt, h1, W_l2, b_l2.reshape(1, D), W_r2, W_post,
      b_post.reshape(1, n_class))
    return out
